# baseline (speedup 1.0000x reference)
import functools

import jax
import jax.numpy as jnp
from jax import lax
from jax.experimental import pallas as pl
from jax.experimental.pallas import tpu as pltpu
from jax.experimental.pallas import tpu_sc as plsc

D_MODEL = 768
VOCAB = 100000
BATCH = 4
SEQ = 8192
SCALE = D_MODEL ** 0.5

_INFO = plsc.get_sparse_core_info()
_NC = _INFO.num_cores
_NS = _INFO.num_subcores
_L = _INFO.num_lanes
_NW = _NC * _NS

_B_TOT = BATCH * SEQ
_PER_W = _B_TOT // _NW
_TPB = SEQ // _PER_W
_C = 64
_NCH = _PER_W // _C
_NBUF = 2
_K = 1

_mesh = plsc.VectorSubcoreMesh(core_axis_name="c", subcore_axis_name="s")


@functools.partial(
    pl.kernel,
    mesh=_mesh,
    out_type=jax.ShapeDtypeStruct((BATCH, SEQ, D_MODEL), jnp.float32),
    scratch_types=[
        pltpu.VMEM((_PER_W,), jnp.int32),
        pltpu.VMEM((_NBUF, _C, D_MODEL), jnp.float32),
    ]
    + [pltpu.SemaphoreType.DMA] * (2 * _NBUF),
)
def _gather_scale(ids_hbm, table_hbm, out_hbm, idx_v, rows_v, *sems):
    gsem = sems[:_NBUF]
    ssem = sems[_NBUF:]
    wid = lax.axis_index("s") * _NC + lax.axis_index("c")
    row = wid // _TPB
    col = (wid % _TPB) * _PER_W
    pltpu.sync_copy(ids_hbm.at[row, pl.ds(col, _PER_W)], idx_v)

    def start_gather(g, b):
        return pltpu.async_copy(
            table_hbm.at[idx_v.at[pl.ds(g * _C, _C)]], rows_v.at[b], gsem[b]
        )

    gather_h = [None] * _NBUF
    store_h = [None] * _NBUF
    for g in range(_K):
        gather_h[g % _NBUF] = start_gather(g, g % _NBUF)
    for g in range(_NCH):
        b = g % _NBUF
        gn = g + _K
        if gn < _NCH:
            bn = gn % _NBUF
            if store_h[bn] is not None:
                store_h[bn].wait()
            gather_h[bn] = start_gather(gn, bn)
        gather_h[b].wait()

        def row_body(r2, carry, b=b):
            for rr in range(2):
                r = r2 * 2 + rr
                for j in range(D_MODEL // _L):
                    sl = pl.ds(j * _L, _L)
                    rows_v[b, r, sl] = rows_v[b, r, sl] * SCALE
            return carry

        lax.fori_loop(0, _C // 2, row_body, 0)
        store_h[b] = pltpu.async_copy(
            rows_v.at[b], out_hbm.at[row, pl.ds(col + g * _C, _C)], ssem[b]
        )
    for b in range(_NBUF):
        if store_h[b] is not None:
            store_h[b].wait()


def kernel(input_ids, emb_table):
    return _gather_scale(input_ids, emb_table)


# speedup vs baseline: 1.1134x; 1.1134x over previous
import functools

import jax
import jax.numpy as jnp
from jax import lax
from jax.experimental import pallas as pl
from jax.experimental.pallas import tpu as pltpu
from jax.experimental.pallas import tpu_sc as plsc

D_MODEL = 768
VOCAB = 100000
BATCH = 4
SEQ = 8192
SCALE = D_MODEL ** 0.5

_INFO = plsc.get_sparse_core_info()
_NC = _INFO.num_cores
_NS = _INFO.num_subcores
_L = _INFO.num_lanes
_NW = _NC * _NS

_B_TOT = BATCH * SEQ
_PER_W = _B_TOT // _NW
_TPB = SEQ // _PER_W
_C = 64
_NCH = _PER_W // _C
_NBUF = 2
_K = 1

_mesh = plsc.VectorSubcoreMesh(core_axis_name="c", subcore_axis_name="s")


@functools.partial(
    pl.kernel,
    mesh=_mesh,
    out_type=jax.ShapeDtypeStruct((BATCH, SEQ, D_MODEL), jnp.float32),
    scratch_types=[
        pltpu.VMEM((_PER_W,), jnp.int32),
        pltpu.VMEM((_NBUF, _C, D_MODEL), jnp.float32),
    ]
    + [pltpu.SemaphoreType.DMA] * (2 * _NBUF),
)
def _gather_scale(ids_hbm, table_hbm, out_hbm, idx_v, rows_v, *sems):
    gsem = sems[:_NBUF]
    ssem = sems[_NBUF:]
    wid = lax.axis_index("s") * _NC + lax.axis_index("c")
    row = wid // _TPB
    col = (wid % _TPB) * _PER_W
    pltpu.sync_copy(ids_hbm.at[row, pl.ds(col, _PER_W)], idx_v)

    def start_gather(g, b):
        return pltpu.async_copy(
            table_hbm.at[idx_v.at[pl.ds(g * _C, _C)]], rows_v.at[b], gsem[b]
        )

    gather_h = [None] * _NBUF
    store_h = [None] * _NBUF
    for g in range(_K):
        gather_h[g % _NBUF] = start_gather(g, g % _NBUF)
    for g in range(_NCH):
        b = g % _NBUF
        gn = g + _K
        if gn < _NCH:
            bn = gn % _NBUF
            if store_h[bn] is not None:
                store_h[bn].wait()
            gather_h[bn] = start_gather(gn, bn)
        gather_h[b].wait()

        @plsc.parallel_loop(0, _C, 1, unroll=2)
        def row_body(r, b=b):
            for j in range(D_MODEL // _L):
                sl = pl.ds(j * _L, _L)
                rows_v[b, r, sl] = rows_v[b, r, sl] * SCALE
        store_h[b] = pltpu.async_copy(
            rows_v.at[b], out_hbm.at[row, pl.ds(col + g * _C, _C)], ssem[b]
        )
    for b in range(_NBUF):
        if store_h[b] is not None:
            store_h[b].wait()


def kernel(input_ids, emb_table):
    return _gather_scale(input_ids, emb_table)


# speedup vs baseline: 1.1366x; 1.0209x over previous
import functools

import jax
import jax.numpy as jnp
from jax import lax
from jax.experimental import pallas as pl
from jax.experimental.pallas import tpu as pltpu
from jax.experimental.pallas import tpu_sc as plsc

D_MODEL = 768
VOCAB = 100000
BATCH = 4
SEQ = 8192
SCALE = D_MODEL ** 0.5

_INFO = plsc.get_sparse_core_info()
_NC = _INFO.num_cores
_NS = _INFO.num_subcores
_L = _INFO.num_lanes
_NW = _NC * _NS

_B_TOT = BATCH * SEQ
_PER_W = _B_TOT // _NW
_TPB = SEQ // _PER_W
_C = 64
_NCH = _PER_W // _C
_NBUF = 2
_K = 1

_mesh = plsc.VectorSubcoreMesh(core_axis_name="c", subcore_axis_name="s")


@functools.partial(
    pl.kernel,
    mesh=_mesh,
    out_type=jax.ShapeDtypeStruct((BATCH, SEQ, D_MODEL), jnp.float32),
    scratch_types=[
        pltpu.VMEM((_PER_W,), jnp.int32),
        pltpu.VMEM((_NBUF, _C, D_MODEL), jnp.float32),
    ]
    + [pltpu.SemaphoreType.DMA] * (3 * _NBUF),
)
def _gather_scale(ids_hbm, table_hbm, out_hbm, idx_v, rows_v, *sems):
    gsem = sems[: 2 * _NBUF]
    ssem = sems[2 * _NBUF :]
    wid = lax.axis_index("s") * _NC + lax.axis_index("c")
    row = wid // _TPB
    col = (wid % _TPB) * _PER_W
    pltpu.sync_copy(ids_hbm.at[row, pl.ds(col, _PER_W)], idx_v)

    _H = _C // 2

    def start_gather_half(g, b, h):
        return pltpu.async_copy(
            table_hbm.at[idx_v.at[pl.ds(g * _C + h * _H, _H)]],
            rows_v.at[b, pl.ds(h * _H, _H)],
            gsem[b * 2 + h],
        )

    def scale_half(b, h):
        def row_body(r, carry):
            for j in range(D_MODEL // _L):
                sl = pl.ds(j * _L, _L)
                rows_v[b, r, sl] = rows_v[b, r, sl] * SCALE
            return carry

        lax.fori_loop(h * _H, (h + 1) * _H, row_body, 0)

    gather_h = [[None, None] for _ in range(_NBUF)]
    store_h = [None] * _NBUF
    gather_h[0][0] = start_gather_half(0, 0, 0)
    gather_h[0][1] = start_gather_half(0, 0, 1)
    for g in range(_NCH):
        b = g % _NBUF
        gn = g + 1
        bn = gn % _NBUF
        if gn < _NCH:
            if store_h[bn] is not None:
                store_h[bn].wait()
            gather_h[bn][0] = start_gather_half(gn, bn, 0)
        gather_h[b][0].wait()
        if gn < _NCH:
            gather_h[bn][1] = start_gather_half(gn, bn, 1)
        scale_half(b, 0)
        gather_h[b][1].wait()
        scale_half(b, 1)
        store_h[b] = pltpu.async_copy(
            rows_v.at[b], out_hbm.at[row, pl.ds(col + g * _C, _C)], ssem[b]
        )
    for b in range(_NBUF):
        if store_h[b] is not None:
            store_h[b].wait()


def kernel(input_ids, emb_table):
    return _gather_scale(input_ids, emb_table)


# speedup vs baseline: 1.1661x; 1.0259x over previous
import functools

import jax
import jax.numpy as jnp
from jax import lax
from jax.experimental import pallas as pl
from jax.experimental.pallas import tpu as pltpu
from jax.experimental.pallas import tpu_sc as plsc

D_MODEL = 768
VOCAB = 100000
BATCH = 4
SEQ = 8192
SCALE = D_MODEL ** 0.5

_INFO = plsc.get_sparse_core_info()
_NC = _INFO.num_cores
_NS = _INFO.num_subcores
_L = _INFO.num_lanes
_NW = _NC * _NS

_B_TOT = BATCH * SEQ
_PER_W = _B_TOT // _NW
_TPB = SEQ // _PER_W
_C = 64
_NCH = _PER_W // _C
_NBUF = 2
_K = 1

_mesh = plsc.VectorSubcoreMesh(core_axis_name="c", subcore_axis_name="s")


@functools.partial(
    pl.kernel,
    mesh=_mesh,
    out_type=jax.ShapeDtypeStruct((BATCH, SEQ, D_MODEL), jnp.float32),
    scratch_types=[
        pltpu.VMEM((_PER_W,), jnp.int32),
        pltpu.VMEM((_NBUF, _C, D_MODEL), jnp.float32),
    ]
    + [pltpu.SemaphoreType.DMA] * (2 * _NBUF),
)
def _gather_scale(ids_hbm, table_hbm, out_hbm, idx_v, rows_v, *sems):
    gsem = sems[:_NBUF]
    ssem = sems[_NBUF:]
    wid = lax.axis_index("s") * _NC + lax.axis_index("c")
    row = wid // _TPB
    col = (wid % _TPB) * _PER_W
    pltpu.sync_copy(ids_hbm.at[row, pl.ds(col, _PER_W)], idx_v)

    def start_gather(g, b):
        return pltpu.async_copy(
            table_hbm.at[idx_v.at[pl.ds(g * _C, _C)]], rows_v.at[b], gsem[b]
        )

    gather_h = [None] * _NBUF
    store_h = [None] * _NBUF
    for g in range(_K):
        gather_h[g % _NBUF] = start_gather(g, g % _NBUF)
    for g in range(_NCH):
        b = g % _NBUF
        gn = g + _K
        if gn < _NCH:
            bn = gn % _NBUF
            if store_h[bn] is not None:
                store_h[bn].wait()
            gather_h[bn] = start_gather(gn, bn)
        gather_h[b].wait()

        def row_body(r, carry, b=b):
            for j in range(D_MODEL // _L):
                sl = pl.ds(j * _L, _L)
                rows_v[b, r, sl] = rows_v[b, r, sl] * SCALE
            return carry

        lax.fori_loop(0, _C, row_body, 0)
        store_h[b] = pltpu.async_copy(
            rows_v.at[b], out_hbm.at[row, pl.ds(col + g * _C, _C)], ssem[b]
        )
    for b in range(_NBUF):
        if store_h[b] is not None:
            store_h[b].wait()


def kernel(input_ids, emb_table):
    return _gather_scale(input_ids, emb_table)
